# baseline (speedup 1.0000x reference)
import jax
import jax.numpy as jnp
from jax import lax
from jax.experimental import pallas as pl
from jax.experimental.pallas import tpu as pltpu
from jax.experimental.pallas import tpu_sc as plsc

B, C, H, W = 4, 384, 224, 224
ROW = H * W
R = B * C

_info = plsc.get_sparse_core_info()
_NC = _info.num_cores
_NS = _info.num_subcores
NW = _NC * _NS
RPW = R // NW


def _sc_body(in_hbm, out_hbm, buf0, buf1, sem0, sem1):
    wid = lax.axis_index("s") * _NC + lax.axis_index("c")
    base = wid * RPW
    b = base // C
    src0 = 2 * b * C + (C - 1) - base

    pltpu.async_copy(in_hbm.at[src0], buf0, sem0)
    pltpu.async_copy(in_hbm.at[src0 - 1], buf1, sem1)

    @pl.loop(0, RPW, step=2)
    def _(g):
        pltpu.make_async_copy(in_hbm.at[src0 - g], buf0, sem0).wait()
        pltpu.sync_copy(buf0, out_hbm.at[base + g])

        @pl.when(g + 2 < RPW)
        def _():
            pltpu.async_copy(in_hbm.at[src0 - (g + 2)], buf0, sem0)

        pltpu.make_async_copy(in_hbm.at[src0 - (g + 1)], buf1, sem1).wait()
        pltpu.sync_copy(buf1, out_hbm.at[base + g + 1])

        @pl.when(g + 3 < RPW)
        def _():
            pltpu.async_copy(in_hbm.at[src0 - (g + 3)], buf1, sem1)


_sc_kernel = pl.kernel(
    _sc_body,
    out_type=jax.ShapeDtypeStruct((R, ROW), jnp.float32),
    mesh=plsc.VectorSubcoreMesh(core_axis_name="c", subcore_axis_name="s"),
    scratch_types=[
        pltpu.VMEM((ROW,), jnp.float32),
        pltpu.VMEM((ROW,), jnp.float32),
        pltpu.SemaphoreType.DMA,
        pltpu.SemaphoreType.DMA,
    ],
)


@jax.jit
def kernel(input):
    flat = input.reshape(R, ROW)
    out = _sc_kernel(flat)
    return out.reshape(B, C, H, W)
